# baseline (speedup 1.0000x reference)
import functools
import math

import jax
import jax.numpy as jnp
from jax import lax
from jax.experimental import pallas as pl
from jax.experimental.pallas import tpu as pltpu
from jax.experimental.pallas import tpu_sc as plsc

SIGMA = 2.0
SEQ = 1024
NNEU = 1024
NBATCH = 8
NROWS = NBATCH * NNEU
NWORDS = SEQ // 4
NW = 32
TILE_ROWS = NROWS // NW
CHUNK = 8192

_mesh = plsc.VectorSubcoreMesh(
    core_axis_name="c", subcore_axis_name="s", num_cores=2, num_subcores=16
)


def _make_sc_hist(nev):
    pieces = [(i, CHUNK) for i in range(0, nev - CHUNK + 1, CHUNK)]
    done = len(pieces) * CHUNK
    if done < nev:
        pieces.append((done, nev - done))

    @functools.partial(
        pl.kernel,
        out_type=jax.ShapeDtypeStruct((NROWS, NWORDS), jnp.int32),
        mesh=_mesh,
        compiler_params=pltpu.CompilerParams(needs_layout_passes=False),
        scratch_types=[
            pltpu.VMEM((CHUNK,), jnp.int32),
            pltpu.VMEM((CHUNK,), jnp.int32),
            pltpu.VMEM((TILE_ROWS, NWORDS), jnp.int32),
            pltpu.SemaphoreType.DMA,
            pltpu.SemaphoreType.DMA,
        ],
    )
    def sc_hist(keys_hbm, out_hbm, kbuf0, kbuf1, hist, sem0, sem1):
        wid = lax.axis_index("s") * 2 + lax.axis_index("c")
        kbufs = (kbuf0, kbuf1)
        sems = (sem0, sem1)
        zvec = jnp.zeros((16,), jnp.int32)
        ones = jnp.ones((16,), jnp.int32)

        row0 = wid * TILE_ROWS
        lo = row0 * SEQ

        sent = jnp.full((16,), -1, jnp.int32)
        for ci, (_, size) in enumerate(pieces):
            if size % 16:
                kbufs[ci % 2][pl.ds(size - size % 16, 16)] = sent

        copies = [None, None]
        copies[0] = pltpu.async_copy(
            keys_hbm.at[pl.ds(0, pieces[0][1])],
            kbufs[0].at[pl.ds(0, pieces[0][1])],
            sems[0],
        )

        @plsc.parallel_loop(0, TILE_ROWS)
        def _(r):
            for j in range(NWORDS // 16):
                hist[r, pl.ds(j * 16, 16)] = zvec

        for ci, (base, size) in enumerate(pieces):
            cur = kbufs[ci % 2]
            copies[ci % 2].wait()
            if ci + 1 < len(pieces):
                nbase, nsize = pieces[ci + 1]
                copies[(ci + 1) % 2] = pltpu.async_copy(
                    keys_hbm.at[pl.ds(nbase, nsize)],
                    kbufs[(ci + 1) % 2].at[pl.ds(0, nsize)],
                    sems[(ci + 1) % 2],
                )

            @plsc.parallel_loop(0, (size + 15) // 16 * 16, step=16, unroll=8)
            def _(g):
                k = cur[pl.ds(g, 16)]
                off = k - lo
                m = plsc.bitcast(off, jnp.uint32) < jnp.uint32(
                    TILE_ROWS * SEQ
                )
                addend = ones << (((off >> 8) & 3) << 3)
                plsc.addupdate_scatter(
                    hist, [off >> 10, off & (NWORDS - 1)], addend, mask=m
                )

        pltpu.sync_copy(hist, out_hbm.at[pl.ds(row0, TILE_ROWS), :])

    return sc_hist


_INV2S2 = 0.5 / (SIGMA * SIGMA)
_NORM = 1.0 / (SIGMA * math.sqrt(2.0 * math.pi))
BM = 4096


_NB = 4
_BW = 320
_STARTS = tuple(min(max(256 * j - 32, 0), SEQ - _BW) for j in range(_NB))


def _conv_body(h_ref, o_ref, g_ref):
    @pl.when(pl.program_id(0) == 0)
    def _():
        r = lax.broadcasted_iota(jnp.int32, (_BW, 256), 0)
        c = lax.broadcasted_iota(jnp.int32, (_BW, 256), 1)
        for j, s in enumerate(_STARTS):
            d = (c + 256 * j - (r + s)).astype(jnp.float32)
            g_ref[:, 256 * j : 256 * (j + 1)] = (
                jnp.exp(-(d * d) * _INV2S2) * _NORM
            ).astype(jnp.bfloat16)

    w = h_ref[...]
    hb = jnp.concatenate(
        [((w >> (8 * b)) & 0xFF).astype(jnp.bfloat16) for b in range(4)],
        axis=1,
    )
    for j, s in enumerate(_STARTS):
        o_ref[:, 256 * j : 256 * (j + 1)] = jnp.dot(
            hb[:, s : s + _BW],
            g_ref[:, 256 * j : 256 * (j + 1)],
            preferred_element_type=jnp.float32,
        )


def _conv(h):
    return pl.pallas_call(
        _conv_body,
        grid=(NROWS // BM,),
        in_specs=[pl.BlockSpec((BM, NWORDS), lambda i: (i, 0))],
        out_specs=pl.BlockSpec((BM, SEQ), lambda i: (i, 0)),
        out_shape=jax.ShapeDtypeStruct((NROWS, SEQ), jnp.float32),
        scratch_shapes=[pltpu.VMEM((_BW, 256 * _NB), jnp.bfloat16)],
        compiler_params=pltpu.CompilerParams(
            dimension_semantics=("arbitrary",)
        ),
    )(h)


def kernel(events, batch_idx):
    nev = events.shape[0]
    keys = (
        batch_idx.astype(jnp.float32) * float(NNEU * SEQ)
        + events[:, 1] * float(SEQ)
        + events[:, 0]
    ).astype(jnp.int32)
    h = _make_sc_hist(nev)(keys)
    out = _conv(h)
    return out.reshape(NBATCH, NNEU, SEQ)


# speedup vs baseline: 1.0109x; 1.0109x over previous
import functools
import math

import jax
import jax.numpy as jnp
from jax import lax
from jax.experimental import pallas as pl
from jax.experimental.pallas import tpu as pltpu
from jax.experimental.pallas import tpu_sc as plsc

SIGMA = 2.0
SEQ = 1024
NNEU = 1024
NBATCH = 8
NROWS = NBATCH * NNEU
NWORDS = SEQ // 8
NW = 32
TILE_ROWS = 512
CHUNK = 8192

_mesh = plsc.VectorSubcoreMesh(
    core_axis_name="c", subcore_axis_name="s", num_cores=2, num_subcores=16
)


def _half_split(size):
    mid = (size // 2 + 15) // 16 * 16
    return (0, mid), (mid, size)


def _make_sc_hist(nev):
    pieces = [(i, CHUNK) for i in range(0, nev - CHUNK + 1, CHUNK)]
    done = len(pieces) * CHUNK
    if done < nev:
        pieces.append((done, nev - done))

    @functools.partial(
        pl.kernel,
        out_type=jax.ShapeDtypeStruct((2, NROWS, NWORDS), jnp.int32),
        mesh=_mesh,
        compiler_params=pltpu.CompilerParams(needs_layout_passes=False),
        scratch_types=[
            pltpu.VMEM((CHUNK,), jnp.int32),
            pltpu.VMEM((CHUNK,), jnp.int32),
            pltpu.VMEM((TILE_ROWS, NWORDS), jnp.int32),
            pltpu.SemaphoreType.DMA,
            pltpu.SemaphoreType.DMA,
        ],
    )
    def sc_hist(keys_hbm, out_hbm, kbuf0, kbuf1, hist, sem0, sem1):
        wid = lax.axis_index("s") * 2 + lax.axis_index("c")
        pair = wid >> 1
        half = wid & 1
        kbufs = (kbuf0, kbuf1)
        sems = (sem0, sem1)
        zvec = jnp.zeros((16,), jnp.int32)
        ones = jnp.ones((16,), jnp.int32)

        row0 = pair * TILE_ROWS
        lo = row0 * SEQ

        sent = jnp.full((16,), -1, jnp.int32)
        for ci, (_, size) in enumerate(pieces):
            if size % 16:
                kbufs[ci % 2][pl.ds(size - size % 16, 16)] = sent

        copies = [None, None]
        copies[0] = pltpu.async_copy(
            keys_hbm.at[pl.ds(0, pieces[0][1])],
            kbufs[0].at[pl.ds(0, pieces[0][1])],
            sems[0],
        )

        @plsc.parallel_loop(0, TILE_ROWS)
        def _(r):
            for j in range(NWORDS // 16):
                hist[r, pl.ds(j * 16, 16)] = zvec

        for ci, (base, size) in enumerate(pieces):
            cur = kbufs[ci % 2]
            copies[ci % 2].wait()
            if ci + 1 < len(pieces):
                nbase, nsize = pieces[ci + 1]
                copies[(ci + 1) % 2] = pltpu.async_copy(
                    keys_hbm.at[pl.ds(nbase, nsize)],
                    kbufs[(ci + 1) % 2].at[pl.ds(0, nsize)],
                    sems[(ci + 1) % 2],
                )

            (s0, e0), (s1, e1) = _half_split((size + 15) // 16 * 16)
            start = jnp.where(half == 0, s0, s1)
            stop = jnp.where(half == 0, e0, e1)

            @plsc.parallel_loop(start, stop, step=16, unroll=8)
            def _(g):
                k = cur[pl.ds(g, 16)]
                off = k - lo
                m = plsc.bitcast(off, jnp.uint32) < jnp.uint32(
                    TILE_ROWS * SEQ
                )
                addend = ones << (((off >> 7) & 7) << 2)
                plsc.addupdate_scatter(
                    hist, [off >> 10, off & (NWORDS - 1)], addend, mask=m
                )

        pltpu.sync_copy(hist, out_hbm.at[half, pl.ds(row0, TILE_ROWS), :])

    return sc_hist


_INV2S2 = 0.5 / (SIGMA * SIGMA)
_NORM = 1.0 / (SIGMA * math.sqrt(2.0 * math.pi))
BM = 2048


_NB = 4
_BW = 320
_STARTS = tuple(min(max(256 * j - 32, 0), SEQ - _BW) for j in range(_NB))


def _conv_body(h_ref, o_ref, g_ref):
    @pl.when(pl.program_id(0) == 0)
    def _():
        r = lax.broadcasted_iota(jnp.int32, (_BW, 256), 0)
        c = lax.broadcasted_iota(jnp.int32, (_BW, 256), 1)
        for j, s in enumerate(_STARTS):
            d = (c + 256 * j - (r + s)).astype(jnp.float32)
            g_ref[:, 256 * j : 256 * (j + 1)] = (
                jnp.exp(-(d * d) * _INV2S2) * _NORM
            ).astype(jnp.bfloat16)

    w = h_ref[0] + h_ref[1]
    hb = jnp.concatenate(
        [((w >> (4 * b)) & 0xF).astype(jnp.bfloat16) for b in range(8)],
        axis=1,
    )
    for j, s in enumerate(_STARTS):
        o_ref[:, 256 * j : 256 * (j + 1)] = jnp.dot(
            hb[:, s : s + _BW],
            g_ref[:, 256 * j : 256 * (j + 1)],
            preferred_element_type=jnp.float32,
        )


def _conv(h):
    return pl.pallas_call(
        _conv_body,
        grid=(NROWS // BM,),
        in_specs=[pl.BlockSpec((2, BM, NWORDS), lambda i: (0, i, 0))],
        out_specs=pl.BlockSpec((BM, SEQ), lambda i: (i, 0)),
        out_shape=jax.ShapeDtypeStruct((NROWS, SEQ), jnp.float32),
        scratch_shapes=[pltpu.VMEM((_BW, 256 * _NB), jnp.bfloat16)],
        compiler_params=pltpu.CompilerParams(
            dimension_semantics=("arbitrary",)
        ),
    )(h)


def kernel(events, batch_idx):
    nev = events.shape[0]
    keys = (
        batch_idx.astype(jnp.float32) * float(NNEU * SEQ)
        + events[:, 1] * float(SEQ)
        + events[:, 0]
    ).astype(jnp.int32)
    h = _make_sc_hist(nev)(keys)
    out = _conv(h)
    return out.reshape(NBATCH, NNEU, SEQ)


# speedup vs baseline: 1.1738x; 1.1612x over previous
import functools
import math

import jax
import jax.numpy as jnp
from jax import lax
from jax.experimental import pallas as pl
from jax.experimental.pallas import tpu as pltpu
from jax.experimental.pallas import tpu_sc as plsc

SIGMA = 2.0
SEQ = 1024
NNEU = 1024
NBATCH = 8
NROWS = NBATCH * NNEU
NWORDS = SEQ // 8
NW = 32
TILE_ROWS = 512
CHUNK = 24576

_mesh = plsc.VectorSubcoreMesh(
    core_axis_name="c", subcore_axis_name="s", num_cores=2, num_subcores=16
)


def _make_sc_hist(nev):
    pieces = []
    base, rem = 0, nev
    while rem >= CHUNK:
        pieces.append((base, CHUNK, True))
        base += CHUNK
        rem -= CHUNK
    if rem >= 32:
        s = rem // 32 * 32
        pieces.append((base, s, True))
        base += s
        rem -= s
    if rem:
        pieces.append((base, rem, False))

    @functools.partial(
        pl.kernel,
        out_type=jax.ShapeDtypeStruct((2, NROWS, NWORDS), jnp.int32),
        mesh=_mesh,
        compiler_params=pltpu.CompilerParams(needs_layout_passes=False),
        scratch_types=[
            pltpu.VMEM((CHUNK // 2,), jnp.int32),
            pltpu.VMEM((CHUNK // 2,), jnp.int32),
            pltpu.VMEM((TILE_ROWS, NWORDS), jnp.int32),
            pltpu.SemaphoreType.DMA,
            pltpu.SemaphoreType.DMA,
        ],
    )
    def sc_hist(keys_hbm, out_hbm, kbuf0, kbuf1, hist, sem0, sem1):
        wid = lax.axis_index("s") * 2 + lax.axis_index("c")
        pair = wid >> 1
        half = wid & 1
        kbufs = (kbuf0, kbuf1)
        sems = (sem0, sem1)
        zvec = jnp.zeros((16,), jnp.int32)
        ones = jnp.ones((16,), jnp.int32)

        row0 = pair * TILE_ROWS
        lo = row0 * SEQ
        sent = jnp.full((16,), -1, jnp.int32)

        def scan(cur, nkeys):
            @plsc.parallel_loop(
                0, (nkeys + 15) // 16 * 16, step=16, unroll=8
            )
            def _(g):
                k = cur[pl.ds(g, 16)]
                off = k - lo
                m = plsc.bitcast(off, jnp.uint32) < jnp.uint32(
                    TILE_ROWS * SEQ
                )
                addend = ones << (((off >> 7) & 7) << 2)
                plsc.addupdate_scatter(
                    hist, [off >> 10, off & (NWORDS - 1)], addend, mask=m
                )

        def issue(ci):
            base, size, split = pieces[ci]
            buf = kbufs[ci % 2]
            if split:
                h = size // 2
                return pltpu.async_copy(
                    keys_hbm.at[pl.ds(base + half * h, h)],
                    buf.at[pl.ds(0, h)],
                    sems[ci % 2],
                )
            if size % 16:
                buf[pl.ds(size - size % 16, 16)] = sent
            return pltpu.async_copy(
                keys_hbm.at[pl.ds(base, size)],
                buf.at[pl.ds(0, size)],
                sems[ci % 2],
            )

        copies = [None, None]
        copies[0] = issue(0)

        @plsc.parallel_loop(0, TILE_ROWS)
        def _(r):
            for j in range(NWORDS // 16):
                hist[r, pl.ds(j * 16, 16)] = zvec

        for ci, (base, size, split) in enumerate(pieces):
            cur = kbufs[ci % 2]
            copies[ci % 2].wait()
            if ci + 1 < len(pieces):
                copies[(ci + 1) % 2] = issue(ci + 1)

            if split:
                scan(cur, size // 2)
            else:

                @pl.when(half == 0)
                def _():
                    scan(cur, size)

        pltpu.sync_copy(hist, out_hbm.at[half, pl.ds(row0, TILE_ROWS), :])

    return sc_hist


_INV2S2 = 0.5 / (SIGMA * SIGMA)
_NORM = 1.0 / (SIGMA * math.sqrt(2.0 * math.pi))
BM = 2048


_NB = 4
_BW = 320
_STARTS = tuple(min(max(256 * j - 32, 0), SEQ - _BW) for j in range(_NB))


def _conv_body(h_ref, o_ref, g_ref):
    @pl.when(pl.program_id(0) == 0)
    def _():
        r = lax.broadcasted_iota(jnp.int32, (_BW, 256), 0)
        c = lax.broadcasted_iota(jnp.int32, (_BW, 256), 1)
        for j, s in enumerate(_STARTS):
            d = (c + 256 * j - (r + s)).astype(jnp.float32)
            g_ref[:, 256 * j : 256 * (j + 1)] = (
                jnp.exp(-(d * d) * _INV2S2) * _NORM
            ).astype(jnp.bfloat16)

    w = h_ref[0] + h_ref[1]
    hb = jnp.concatenate(
        [((w >> (4 * b)) & 0xF).astype(jnp.bfloat16) for b in range(8)],
        axis=1,
    )
    for j, s in enumerate(_STARTS):
        o_ref[:, 256 * j : 256 * (j + 1)] = jnp.dot(
            hb[:, s : s + _BW],
            g_ref[:, 256 * j : 256 * (j + 1)],
            preferred_element_type=jnp.float32,
        )


def _conv(h):
    return pl.pallas_call(
        _conv_body,
        grid=(NROWS // BM,),
        in_specs=[pl.BlockSpec((2, BM, NWORDS), lambda i: (0, i, 0))],
        out_specs=pl.BlockSpec((BM, SEQ), lambda i: (i, 0)),
        out_shape=jax.ShapeDtypeStruct((NROWS, SEQ), jnp.float32),
        scratch_shapes=[pltpu.VMEM((_BW, 256 * _NB), jnp.bfloat16)],
        compiler_params=pltpu.CompilerParams(
            dimension_semantics=("arbitrary",)
        ),
    )(h)


def kernel(events, batch_idx):
    nev = events.shape[0]
    keys = (
        batch_idx.astype(jnp.float32) * float(NNEU * SEQ)
        + events[:, 1] * float(SEQ)
        + events[:, 0]
    ).astype(jnp.int32)
    h = _make_sc_hist(nev)(keys)
    out = _conv(h)
    return out.reshape(NBATCH, NNEU, SEQ)
